# Initial kernel scaffold; baseline (speedup 1.0000x reference)
import functools

import jax
import jax.numpy as jnp
from jax import lax
from jax.experimental import pallas as pl
from jax.experimental.pallas import tpu as pltpu
from jax.experimental.pallas import tpu_sc as plsc

N = 10000
E = 320000
FEAT = 128
F = 64
NP = 10240
NT = 16
EC = E // NT
BB = 80
NB = EC // BB
GROUPS = BB // 16
DROWS = NP // 16


def _rsqrt16(v):
    bits = plsc.bitcast(v, jnp.int32)
    y = plsc.bitcast(jnp.full((16,), 0x5F3759DF, jnp.int32) - (bits >> 1),
                     jnp.float32)
    for _ in range(3):
        y = y * (1.5 - 0.5 * v * y * y)
    return y


def _splat(v16, j):
    return jnp.take(v16, jnp.full((16,), j, jnp.int32),
                    mode="promise_in_bounds")


def _sc_body(ei, ew, xh, tx1_out, s2_out,
             tx1_sh, s2_sh, deg_sh, dinv_sh,
             dinv_loc, rows, srcb, dstb, ewb, normc, oneh, degrow,
             degloc, zbuf, sem):
    c = lax.axis_index("c")
    s = lax.axis_index("s")
    z16 = jnp.zeros((16,), jnp.float32)
    iota = lax.iota(jnp.int32, 16)

    for i in range(BB):
        for q in range(F // 16):
            zbuf[i, pl.ds(q * 16, 16)] = z16
    r0 = s * (NP // NT)
    for i in range(NP // NT // 64):
        pltpu.sync_copy(zbuf.at[pl.ds(0, 64)], tx1_sh.at[pl.ds(r0 + i * 64, 64)])
        pltpu.sync_copy(zbuf.at[pl.ds(0, 64)], s2_sh.at[pl.ds(r0 + i * 64, 64)])
    pltpu.sync_copy(zbuf.at[pl.ds(0, DROWS // NT), pl.ds(0, 16)],
                    deg_sh.at[pl.ds(s * (DROWS // NT), DROWS // NT)])
    pltpu.sync_copy(zbuf.at[pl.ds(0, BB), pl.ds(0, 16)], oneh)
    plsc.subcore_barrier()

    def deg_body(b, carry):
        off = s * EC + b * BB
        pltpu.sync_copy(ei.at[0].at[pl.ds(off, BB)], srcb)
        pltpu.sync_copy(ew.at[pl.ds(off, BB)], ewb)
        cols = []
        for g in range(GROUPS):
            s16 = srcb[pl.ds(g * 16, 16)]
            e16 = ewb[pl.ds(g * 16, 16)]
            col = s16 & 15
            degrow[pl.ds(g * 16, 16)] = s16 >> 4
            plsc.store_scatter(oneh, [iota + g * 16, col], e16)
            cols.append(col)
        pltpu.sync_copy(oneh, deg_sh.at[degrow], add=True)
        for g in range(GROUPS):
            plsc.store_scatter(oneh, [iota + g * 16, cols[g]], z16)
        return carry

    lax.fori_loop(0, NB, deg_body, 0)
    plsc.subcore_barrier()

    dr = DROWS // NT
    pltpu.sync_copy(deg_sh.at[pl.ds(s * dr, dr)], degloc)
    for i in range(dr):
        d16 = degloc[i, pl.ds(0, 16)]
        y = _rsqrt16(jnp.maximum(d16, 1e-12))
        degloc[i, pl.ds(0, 16)] = jnp.where(d16 > 0, y, 0.0)
    pltpu.sync_copy(degloc, dinv_sh.at[pl.ds(s * dr, dr)])
    plsc.subcore_barrier()
    pltpu.sync_copy(dinv_sh, dinv_loc)

    def hop1_body(b, carry):
        off = s * EC + b * BB
        pltpu.sync_copy(ei.at[0].at[pl.ds(off, BB)], srcb)
        pltpu.sync_copy(ei.at[1].at[pl.ds(off, BB)], dstb)
        pltpu.sync_copy(ew.at[pl.ds(off, BB)], ewb)
        norms = []
        for g in range(GROUPS):
            s16 = srcb[pl.ds(g * 16, 16)]
            d16 = dstb[pl.ds(g * 16, 16)]
            e16 = ewb[pl.ds(g * 16, 16)]
            dv_s = plsc.load_gather(dinv_loc, [s16 >> 4, s16 & 15])
            dv_d = plsc.load_gather(dinv_loc, [d16 >> 4, d16 & 15])
            n16 = -(dv_s * e16 * dv_d)
            normc[pl.ds(b * BB + g * 16, 16)] = n16
            norms.append(n16)
        pltpu.async_copy(xh.at[c].at[srcb], rows, sem).wait()
        for g in range(GROUPS):
            for j in range(16):
                sc = _splat(norms[g], j)
                e = g * 16 + j
                for q in range(F // 16):
                    rows[e, pl.ds(q * 16, 16)] = rows[e, pl.ds(q * 16, 16)] * sc
        pltpu.sync_copy(rows, tx1_sh.at[dstb], add=True)
        return carry

    lax.fori_loop(0, NB, hop1_body, 0)
    plsc.subcore_barrier()

    def hop2_body(b, carry):
        off = s * EC + b * BB
        pltpu.sync_copy(ei.at[0].at[pl.ds(off, BB)], srcb)
        pltpu.sync_copy(ei.at[1].at[pl.ds(off, BB)], dstb)
        pltpu.async_copy(tx1_sh.at[srcb], rows, sem).wait()
        for g in range(GROUPS):
            n16 = normc[pl.ds(b * BB + g * 16, 16)]
            for j in range(16):
                sc = _splat(n16, j)
                e = g * 16 + j
                for q in range(F // 16):
                    rows[e, pl.ds(q * 16, 16)] = rows[e, pl.ds(q * 16, 16)] * sc
        pltpu.sync_copy(rows, s2_sh.at[dstb], add=True)
        return carry

    lax.fori_loop(0, NB, hop2_body, 0)
    plsc.subcore_barrier()

    rows_per_tile = NP // NT
    pltpu.sync_copy(tx1_sh.at[pl.ds(r0, rows_per_tile)],
                    tx1_out.at[c].at[pl.ds(r0, rows_per_tile)])
    pltpu.sync_copy(s2_sh.at[pl.ds(r0, rows_per_tile)],
                    s2_out.at[c].at[pl.ds(r0, rows_per_tile)])


def _sparse_hops(ei, ew, xh):
    mesh = plsc.VectorSubcoreMesh(core_axis_name="c", subcore_axis_name="s")
    f32 = jnp.float32
    kern = pl.kernel(
        _sc_body,
        out_type=[jax.ShapeDtypeStruct((2, NP, F), f32),
                  jax.ShapeDtypeStruct((2, NP, F), f32)],
        mesh=mesh,
        scratch_types=[
            pltpu.VMEM_SHARED((NP, F), f32),
            pltpu.VMEM_SHARED((NP, F), f32),
            pltpu.VMEM_SHARED((DROWS, 16), f32),
            pltpu.VMEM_SHARED((DROWS, 16), f32),
            pltpu.VMEM((DROWS, 16), f32),
            pltpu.VMEM((BB, F), f32),
            pltpu.VMEM((BB,), jnp.int32),
            pltpu.VMEM((BB,), jnp.int32),
            pltpu.VMEM((BB,), f32),
            pltpu.VMEM((EC,), f32),
            pltpu.VMEM((BB, 16), f32),
            pltpu.VMEM((BB,), jnp.int32),
            pltpu.VMEM((DROWS // NT, 16), f32),
            pltpu.VMEM((BB, F), f32),
            pltpu.SemaphoreType.DMA,
        ],
    )
    return kern(ei, ew, xh)


def _dense_body(x_ref, t1_ref, s2_ref, wzh_ref, bzh_ref, wlin_ref, blin_ref,
                out_ref):
    xb = x_ref[...]
    t1 = jnp.concatenate([t1_ref[0], t1_ref[1]], axis=1)
    s2 = jnp.concatenate([s2_ref[0], s2_ref[1]], axis=1)
    tx2 = 2.0 * s2 - xb
    xt = jnp.concatenate([xb, t1, tx2], axis=1)
    a = jnp.dot(xt, wzh_ref[...], preferred_element_type=jnp.float32)
    a = a + bzh_ref[...]
    z = jax.nn.sigmoid(a[:, :FEAT])
    ht = jnp.tanh(a[:, FEAT:])
    h = jnp.maximum((1.0 - z) * ht, 0.0)
    out_ref[...] = (jnp.dot(h, wlin_ref[...], preferred_element_type=jnp.float32)
                    + blin_ref[...])


def _dense(x, tx1h, s2h, wzh, bzh, wlin, blin):
    R = 512
    return pl.pallas_call(
        _dense_body,
        grid=(NP // R,),
        in_specs=[
            pl.BlockSpec((R, FEAT), lambda i: (i, 0)),
            pl.BlockSpec((2, R, F), lambda i: (0, i, 0)),
            pl.BlockSpec((2, R, F), lambda i: (0, i, 0)),
            pl.BlockSpec((3 * FEAT, 2 * FEAT), lambda i: (0, 0)),
            pl.BlockSpec((1, 2 * FEAT), lambda i: (0, 0)),
            pl.BlockSpec((FEAT, FEAT), lambda i: (0, 0)),
            pl.BlockSpec((1, FEAT), lambda i: (0, 0)),
        ],
        out_specs=pl.BlockSpec((R, FEAT), lambda i: (i, 0)),
        out_shape=jax.ShapeDtypeStruct((N, FEAT), jnp.float32),
    )(x, tx1h, s2h, wzh, bzh, wlin, blin)


def kernel(x, edge_index, edge_weight, Wxz, bxz, Whz, bhz, Wxr, bxr, Whr, bhr,
           Wxh, bxh, Whh, bhh, Wlin, blin):
    xh = jnp.stack([x[:, :F], x[:, F:]])
    tx1h, s2h = _sparse_hops(edge_index, edge_weight, xh)
    wzh = jnp.concatenate([Wxz.reshape(3 * FEAT, FEAT),
                           Wxh.reshape(3 * FEAT, FEAT)], axis=1)
    bzh = jnp.concatenate([bxz + bhz, bxh + bhh]).reshape(1, 2 * FEAT)
    return _dense(x, tx1h, s2h, wzh, bzh, Wlin, blin.reshape(1, FEAT))


# speedup vs baseline: 14.2957x; 14.2957x over previous
import jax
import jax.numpy as jnp
from jax import lax
from jax.experimental import pallas as pl
from jax.experimental.pallas import tpu as pltpu
from jax.experimental.pallas import tpu_sc as plsc

N = 10000
E = 320000
FEAT = 128
NP = 10240
NC = 2
NT = 16
NW = NC * NT
BB = 128
TOTB = E // BB
GROUPS = BB // 16
DVT = NP // NT
VREGS = FEAT // 16


def _rsqrt16(v):
    bits = lax.bitcast_convert_type(v, jnp.int32)
    y = lax.bitcast_convert_type(
        jnp.full((16,), 0x5F3759DF, jnp.int32) - (bits >> 1), jnp.float32)
    for _ in range(3):
        y = y * (1.5 - 0.5 * v * y * y)
    return y


def _splat(v16, j):
    return v16.at[jnp.full((16,), j, jnp.int32)].get(mode="promise_in_bounds")


def _scale_rows(rows, norms):
    for g in range(GROUPS):
        for j in range(16):
            sc = _splat(norms[g], j)
            e = g * 16 + j
            for q in range(VREGS):
                rows[e, pl.ds(q * 16, 16)] = rows[e, pl.ds(q * 16, 16)] * sc


def _zero_shared(acc_sh, zbuf, r0):
    z16 = jnp.zeros((16,), jnp.float32)
    for i in range(64):
        for q in range(VREGS):
            zbuf[i, pl.ds(q * 16, 16)] = z16
    for i in range(NP // NT // 64):
        pltpu.sync_copy(zbuf, acc_sh.at[pl.ds(r0 + i * 64, 64)])


def _hop1_body(ei, ew, x, tx1_out, norm_out,
               tx1_sh, deg_sh, dinv_sh, dinv_loc, rows, srcb, dstb, ewb,
               normb, degloc, zbuf, sem):
    c = lax.axis_index("c")
    s = lax.axis_index("s")
    wid = s * NC + c
    r0 = s * (NP // NT)

    _zero_shared(tx1_sh, zbuf, r0)
    z16 = jnp.zeros((16,), jnp.float32)
    for i in range(DVT // 16):
        degloc[pl.ds(i * 16, 16)] = z16
    pltpu.sync_copy(degloc, deg_sh.at[pl.ds(s * DVT, DVT)])
    plsc.subcore_barrier()

    def deg_body(k, carry):
        off = (s + k * NT) * BB
        pltpu.sync_copy(ew.at[pl.ds(off, BB)], ewb)
        pltpu.sync_copy(ei.at[0].at[pl.ds(off, BB)], srcb)
        pltpu.sync_copy(ewb, deg_sh.at[srcb], add=True)
        return carry

    lax.fori_loop(0, (TOTB - s + NT - 1) // NT, deg_body, 0)
    plsc.subcore_barrier()

    pltpu.sync_copy(deg_sh.at[pl.ds(s * DVT, DVT)], degloc)
    for i in range(DVT // 16):
        d16 = degloc[pl.ds(i * 16, 16)]
        y = _rsqrt16(jnp.maximum(d16, 1e-12))
        degloc[pl.ds(i * 16, 16)] = jnp.where(d16 > 0, y, 0.0)
    pltpu.sync_copy(degloc, dinv_sh.at[pl.ds(s * DVT, DVT)])
    plsc.subcore_barrier()
    pltpu.sync_copy(dinv_sh, dinv_loc)

    def hop_body(k, carry):
        off = (wid + k * NW) * BB
        pltpu.sync_copy(ei.at[0].at[pl.ds(off, BB)], srcb)
        pltpu.sync_copy(ei.at[1].at[pl.ds(off, BB)], dstb)
        pltpu.sync_copy(ew.at[pl.ds(off, BB)], ewb)
        norms = []
        for g in range(GROUPS):
            s16 = srcb[pl.ds(g * 16, 16)]
            d16 = dstb[pl.ds(g * 16, 16)]
            e16 = ewb[pl.ds(g * 16, 16)]
            dv_s = plsc.load_gather(dinv_loc, [s16])
            dv_d = plsc.load_gather(dinv_loc, [d16])
            n16 = -(dv_s * e16 * dv_d)
            normb[pl.ds(g * 16, 16)] = n16
            norms.append(n16)
        pltpu.sync_copy(normb, norm_out.at[pl.ds(off, BB)])
        pltpu.async_copy(x.at[srcb], rows, sem).wait()
        _scale_rows(rows, norms)
        pltpu.sync_copy(rows, tx1_sh.at[dstb], add=True)
        return carry

    lax.fori_loop(0, (TOTB - wid + NW - 1) // NW, hop_body, 0)
    plsc.subcore_barrier()

    pltpu.sync_copy(tx1_sh.at[pl.ds(r0, NP // NT)],
                    tx1_out.at[c].at[pl.ds(r0, NP // NT)])


def _hop2_body(ei, nrm, tx1, s2_out,
               s2_sh, rows, srcb, dstb, normb, zbuf, sem):
    c = lax.axis_index("c")
    s = lax.axis_index("s")
    wid = s * NC + c
    r0 = s * (NP // NT)

    _zero_shared(s2_sh, zbuf, r0)
    plsc.subcore_barrier()

    def hop_body(k, carry):
        off = (wid + k * NW) * BB
        pltpu.sync_copy(ei.at[0].at[pl.ds(off, BB)], srcb)
        pltpu.sync_copy(ei.at[1].at[pl.ds(off, BB)], dstb)
        pltpu.sync_copy(nrm.at[pl.ds(off, BB)], normb)
        norms = [normb[pl.ds(g * 16, 16)] for g in range(GROUPS)]
        pltpu.async_copy(tx1.at[srcb], rows, sem).wait()
        _scale_rows(rows, norms)
        pltpu.sync_copy(rows, s2_sh.at[dstb], add=True)
        return carry

    lax.fori_loop(0, (TOTB - wid + NW - 1) // NW, hop_body, 0)
    plsc.subcore_barrier()

    pltpu.sync_copy(s2_sh.at[pl.ds(r0, NP // NT)],
                    s2_out.at[c].at[pl.ds(r0, NP // NT)])


def _sc_mesh():
    return plsc.VectorSubcoreMesh(core_axis_name="c", subcore_axis_name="s")


def _hop1(ei, ew, x):
    f32 = jnp.float32
    kern = pl.kernel(
        _hop1_body,
        out_type=[jax.ShapeDtypeStruct((NC, NP, FEAT), f32),
                  jax.ShapeDtypeStruct((E,), f32)],
        mesh=_sc_mesh(),
        compiler_params=pltpu.CompilerParams(needs_layout_passes=False),
        scratch_types=[
            pltpu.VMEM_SHARED((NP, FEAT), f32),
            pltpu.VMEM_SHARED((NP,), f32),
            pltpu.VMEM_SHARED((NP,), f32),
            pltpu.VMEM((NP,), f32),
            pltpu.VMEM((BB, FEAT), f32),
            pltpu.VMEM((BB,), jnp.int32),
            pltpu.VMEM((BB,), jnp.int32),
            pltpu.VMEM((BB,), f32),
            pltpu.VMEM((BB,), f32),
            pltpu.VMEM((DVT,), f32),
            pltpu.VMEM((64, FEAT), f32),
            pltpu.SemaphoreType.DMA,
        ],
    )
    return kern(ei, ew, x)


def _hop2(ei, nrm, tx1):
    f32 = jnp.float32
    kern = pl.kernel(
        _hop2_body,
        out_type=[jax.ShapeDtypeStruct((NC, NP, FEAT), f32)],
        mesh=_sc_mesh(),
        compiler_params=pltpu.CompilerParams(needs_layout_passes=False),
        scratch_types=[
            pltpu.VMEM_SHARED((NP, FEAT), f32),
            pltpu.VMEM((BB, FEAT), f32),
            pltpu.VMEM((BB,), jnp.int32),
            pltpu.VMEM((BB,), jnp.int32),
            pltpu.VMEM((BB,), f32),
            pltpu.VMEM((64, FEAT), f32),
            pltpu.SemaphoreType.DMA,
        ],
    )
    return kern(ei, nrm, tx1)[0]


def _sum_body(p_ref, out_ref):
    out_ref[...] = p_ref[0] + p_ref[1]


def _sum_partials(p):
    R = 1024
    return pl.pallas_call(
        _sum_body,
        grid=(NP // R,),
        in_specs=[pl.BlockSpec((NC, R, FEAT), lambda i: (0, i, 0))],
        out_specs=pl.BlockSpec((R, FEAT), lambda i: (i, 0)),
        out_shape=jax.ShapeDtypeStruct((NP, FEAT), jnp.float32),
    )(p)


def _dense_body(x_ref, t1_ref, s2_ref, wzh_ref, bzh_ref, wlin_ref, blin_ref,
                out_ref):
    xb = x_ref[...]
    t1 = t1_ref[...]
    s2 = s2_ref[0] + s2_ref[1]
    tx2 = 2.0 * s2 - xb
    xt = jnp.concatenate([xb, t1, tx2], axis=1)
    a = jnp.dot(xt, wzh_ref[...], preferred_element_type=jnp.float32)
    a = a + bzh_ref[...]
    z = jax.nn.sigmoid(a[:, :FEAT])
    ht = jnp.tanh(a[:, FEAT:])
    h = jnp.maximum((1.0 - z) * ht, 0.0)
    out_ref[...] = (jnp.dot(h, wlin_ref[...], preferred_element_type=jnp.float32)
                    + blin_ref[...])


def _dense(x, tx1, s2p, wzh, bzh, wlin, blin):
    R = 512
    return pl.pallas_call(
        _dense_body,
        grid=(NP // R,),
        in_specs=[
            pl.BlockSpec((R, FEAT), lambda i: (i, 0)),
            pl.BlockSpec((R, FEAT), lambda i: (i, 0)),
            pl.BlockSpec((NC, R, FEAT), lambda i: (0, i, 0)),
            pl.BlockSpec((3 * FEAT, 2 * FEAT), lambda i: (0, 0)),
            pl.BlockSpec((1, 2 * FEAT), lambda i: (0, 0)),
            pl.BlockSpec((FEAT, FEAT), lambda i: (0, 0)),
            pl.BlockSpec((1, FEAT), lambda i: (0, 0)),
        ],
        out_specs=pl.BlockSpec((R, FEAT), lambda i: (i, 0)),
        out_shape=jax.ShapeDtypeStruct((N, FEAT), jnp.float32),
    )(x, tx1, s2p, wzh, bzh, wlin, blin)


def kernel(x, edge_index, edge_weight, Wxz, bxz, Whz, bhz, Wxr, bxr, Whr, bhr,
           Wxh, bxh, Whh, bhh, Wlin, blin):
    tx1p, nrm = _hop1(edge_index, edge_weight, x)
    tx1 = _sum_partials(tx1p)
    s2p = _hop2(edge_index, nrm, tx1)
    wzh = jnp.concatenate([Wxz.reshape(3 * FEAT, FEAT),
                           Wxh.reshape(3 * FEAT, FEAT)], axis=1)
    bzh = jnp.concatenate([bxz + bhz, bxh + bhh]).reshape(1, 2 * FEAT)
    return _dense(x, tx1[:N], s2p[:, :N], wzh, bzh, Wlin, blin.reshape(1, FEAT))
